# Initial kernel scaffold; baseline (speedup 1.0000x reference)
import jax
import jax.numpy as jnp
from jax.experimental import pallas as pl


def kernel(x, pos_table):
    raise NotImplementedError("write your pallas kernel here")


# speedup vs baseline: 1.4374x; 1.4374x over previous
import jax
import jax.numpy as jnp
from jax.experimental import pallas as pl

BLK = 512


def _add_kernel(x_ref, pos_ref, o_ref):
    o_ref[...] = x_ref[...] + pos_ref[...]


def kernel(x, pos_table):
    batch, maxlen, dim = x.shape
    seq_blocks = maxlen // BLK
    return pl.pallas_call(
        _add_kernel,
        grid=(seq_blocks, batch),
        in_specs=[
            pl.BlockSpec((1, BLK, dim), lambda j, i: (i, j, 0)),
            pl.BlockSpec((BLK, dim), lambda j, i: (j, 0)),
        ],
        out_specs=pl.BlockSpec((1, BLK, dim), lambda j, i: (i, j, 0)),
        out_shape=jax.ShapeDtypeStruct(x.shape, x.dtype),
    )(x, pos_table)


# speedup vs baseline: 1.7954x; 1.2490x over previous
import jax
import jax.numpy as jnp
from jax.experimental import pallas as pl

BLK = 2048


def _add_kernel(x_ref, pos_ref, o_ref):
    o_ref[...] = x_ref[...] + pos_ref[...]


def kernel(x, pos_table):
    batch, maxlen, dim = x.shape
    seq_blocks = maxlen // BLK
    return pl.pallas_call(
        _add_kernel,
        grid=(seq_blocks, batch),
        in_specs=[
            pl.BlockSpec((1, BLK, dim), lambda j, i: (i, j, 0)),
            pl.BlockSpec((BLK, dim), lambda j, i: (j, 0)),
        ],
        out_specs=pl.BlockSpec((1, BLK, dim), lambda j, i: (i, j, 0)),
        out_shape=jax.ShapeDtypeStruct(x.shape, x.dtype),
    )(x, pos_table)


# speedup vs baseline: 1.7976x; 1.0012x over previous
import jax
import jax.numpy as jnp
from jax.experimental import pallas as pl
from jax.experimental.pallas import tpu as pltpu

BLK = 2048


def _add_kernel(x_ref, pos_ref, o_ref):
    o_ref[...] = x_ref[...] + pos_ref[...]


def kernel(x, pos_table):
    batch, maxlen, dim = x.shape
    seq_blocks = maxlen // BLK
    return pl.pallas_call(
        _add_kernel,
        grid=(seq_blocks, batch),
        in_specs=[
            pl.BlockSpec((1, BLK, dim), lambda j, i: (i, j, 0)),
            pl.BlockSpec((BLK, dim), lambda j, i: (j, 0)),
        ],
        out_specs=pl.BlockSpec((1, BLK, dim), lambda j, i: (i, j, 0)),
        out_shape=jax.ShapeDtypeStruct(x.shape, x.dtype),
        compiler_params=pltpu.CompilerParams(
            dimension_semantics=("parallel", "parallel"),
        ),
    )(x, pos_table)
